# baseline (speedup 1.0000x reference)
import functools

import jax
import jax.numpy as jnp
from jax import lax
from jax.experimental import pallas as pl
from jax.experimental.pallas import tpu as pltpu
from jax.experimental.pallas import tpu_sc as plsc

VOCAB = 100000
CTX = 512
D = 8
DP = 16

_NC, _NS = 2, 16
_NW = _NC * _NS
_TOK_PER_W = CTX // _NW

VC = 2048
_STRIDE = 102400


def _tc_detile(table_t):
    def body(t_ref, o_ref):
        for dd in range(D):
            o_ref[pl.ds(dd * _STRIDE, VOCAB)] = t_ref[dd]

    return pl.pallas_call(
        body,
        in_specs=[pl.BlockSpec((D, VOCAB), lambda: (0, 0))],
        out_specs=pl.BlockSpec((D * _STRIDE,), lambda: (0,)),
        out_shape=jax.ShapeDtypeStruct((D * _STRIDE,), jnp.float32),
    )(table_t)


def _sc_embed_t(flat_t, x):
    mesh = plsc.VectorSubcoreMesh(core_axis_name="c", subcore_axis_name="s")

    @functools.partial(
        pl.kernel,
        mesh=mesh,
        out_type=jax.ShapeDtypeStruct((D + 1, CTX), jnp.float32),
        scratch_types=[
            pltpu.VMEM((_TOK_PER_W,), jnp.int32),
            pltpu.VMEM((D * _TOK_PER_W,), jnp.int32),
            pltpu.VMEM((D * _TOK_PER_W,), jnp.float32),
            pltpu.VMEM((_TOK_PER_W,), jnp.float32),
            pltpu.SemaphoreType.DMA,
        ],
        compiler_params=pltpu.CompilerParams(
            use_tc_tiling_on_sc=False, needs_layout_passes=False
        ),
    )
    def k(flat_hbm, idx_hbm, gt_hbm, idx_v, iall_v, g_v, ones_v, sem):
        wid = lax.axis_index("s") * _NC + lax.axis_index("c")
        tbase = wid * _TOK_PER_W
        pltpu.sync_copy(idx_hbm.at[pl.ds(tbase, _TOK_PER_W)], idx_v)
        xv = idx_v[...]
        for dd in range(D):
            iall_v[pl.ds(dd * _TOK_PER_W, _TOK_PER_W)] = xv + dd * _STRIDE
        pltpu.async_copy(flat_hbm.at[iall_v], g_v, sem).wait()
        ones_v[...] = jnp.full((_TOK_PER_W,), 1.0, jnp.float32)
        for dd in range(D):
            pltpu.sync_copy(
                g_v.at[pl.ds(dd * _TOK_PER_W, _TOK_PER_W)],
                gt_hbm.at[dd, pl.ds(tbase, _TOK_PER_W)],
            )
        pltpu.sync_copy(ones_v, gt_hbm.at[D, pl.ds(tbase, _TOK_PER_W)])

    return k(flat_t, x)


def _tc_unembed_t(wt, b, gt, post):
    def body(wt_ref, b_ref, gt_ref, pt_ref, o_ref):
        lhs = jnp.concatenate([wt_ref[...], b_ref[...][None, :]], axis=0)
        pos9 = jnp.concatenate(
            [pt_ref[...], jnp.zeros((1, CTX), jnp.float32)], axis=0
        )
        rhs = gt_ref[...] + pos9
        o_ref[...] = lax.dot_general(
            lhs,
            rhs,
            dimension_numbers=(((0,), (0,)), ((), ())),
            preferred_element_type=jnp.float32,
        )

    return pl.pallas_call(
        body,
        grid=(pl.cdiv(VOCAB, VC),),
        in_specs=[
            pl.BlockSpec((D, VC), lambda i: (0, i)),
            pl.BlockSpec((VC,), lambda i: (i,)),
            pl.BlockSpec((D + 1, CTX), lambda i: (0, 0)),
            pl.BlockSpec((D, CTX), lambda i: (0, 0)),
        ],
        out_specs=pl.BlockSpec((VC, CTX), lambda i: (i, 0)),
        out_shape=jax.ShapeDtypeStruct((VOCAB, CTX), jnp.float32),
    )(wt, b, gt, post)


def kernel(x, info_embedding, position_embedding, W, b):
    flat_t = _tc_detile(info_embedding.T)
    gt = _sc_embed_t(flat_t, x)
    out_t = _tc_unembed_t(W.T, b, gt, position_embedding.T)
    return out_t.T


# speedup vs baseline: 1.0884x; 1.0884x over previous
import functools

import jax
import jax.numpy as jnp
from jax import lax
from jax.experimental import pallas as pl
from jax.experimental.pallas import tpu as pltpu
from jax.experimental.pallas import tpu_sc as plsc

VOCAB = 100000
CTX = 512
D = 8

_NC, _NS = 2, 16
_NW = _NC * _NS
_TOK_PER_W = CTX // _NW

VC = 4096
_STRIDE = 102400


def _tc_detile(table_t):
    def body(t_ref, o_ref):
        for dd in range(D):
            o_ref[pl.ds(dd * _STRIDE, VOCAB)] = t_ref[dd]

    return pl.pallas_call(
        body,
        in_specs=[pl.BlockSpec((D, VOCAB), lambda: (0, 0))],
        out_specs=pl.BlockSpec((D * _STRIDE,), lambda: (0,)),
        out_shape=jax.ShapeDtypeStruct((D * _STRIDE,), jnp.float32),
    )(table_t)


def _sc_embed_t(flat_t, x):
    mesh = plsc.VectorSubcoreMesh(core_axis_name="c", subcore_axis_name="s")

    @functools.partial(
        pl.kernel,
        mesh=mesh,
        out_type=jax.ShapeDtypeStruct((D + 1, CTX), jnp.float32),
        scratch_types=[
            pltpu.VMEM((_TOK_PER_W,), jnp.int32),
            pltpu.VMEM((D * _TOK_PER_W,), jnp.int32),
            pltpu.VMEM((D * _TOK_PER_W,), jnp.float32),
            pltpu.VMEM((_TOK_PER_W,), jnp.float32),
            pltpu.SemaphoreType.DMA,
        ],
        compiler_params=pltpu.CompilerParams(
            use_tc_tiling_on_sc=False, needs_layout_passes=False
        ),
    )
    def k(flat_hbm, idx_hbm, gt_hbm, idx_v, iall_v, g_v, ones_v, sem):
        wid = lax.axis_index("s") * _NC + lax.axis_index("c")
        tbase = wid * _TOK_PER_W
        pltpu.sync_copy(idx_hbm.at[pl.ds(tbase, _TOK_PER_W)], idx_v)
        xv = idx_v[...]
        for dd in range(D):
            iall_v[pl.ds(dd * _TOK_PER_W, _TOK_PER_W)] = xv + dd * _STRIDE
        pltpu.async_copy(flat_hbm.at[iall_v], g_v, sem).wait()
        ones_v[...] = jnp.full((_TOK_PER_W,), 1.0, jnp.float32)
        for dd in range(D):
            pltpu.sync_copy(
                g_v.at[pl.ds(dd * _TOK_PER_W, _TOK_PER_W)],
                gt_hbm.at[dd, pl.ds(tbase, _TOK_PER_W)],
            )
        pltpu.sync_copy(ones_v, gt_hbm.at[D, pl.ds(tbase, _TOK_PER_W)])

    return k(flat_t, x)


def _tc_unembed_t(wt, b, gt, post):
    def body(wt_ref, b_ref, gt_ref, pt_ref, o_ref):
        lhs = jnp.concatenate([wt_ref[...], b_ref[...][None, :]], axis=0)
        pos9 = jnp.concatenate(
            [pt_ref[...], jnp.zeros((1, CTX), jnp.float32)], axis=0
        )
        rhs = gt_ref[...] + pos9
        o_ref[...] = lax.dot_general(
            lhs,
            rhs,
            dimension_numbers=(((0,), (0,)), ((), ())),
            preferred_element_type=jnp.float32,
        )

    return pl.pallas_call(
        body,
        grid=(pl.cdiv(VOCAB, VC),),
        in_specs=[
            pl.BlockSpec((D, VC), lambda i: (0, i)),
            pl.BlockSpec((VC,), lambda i: (i,)),
            pl.BlockSpec((D + 1, CTX), lambda i: (0, 0)),
            pl.BlockSpec((D, CTX), lambda i: (0, 0)),
        ],
        out_specs=pl.BlockSpec((VC, CTX), lambda i: (i, 0)),
        out_shape=jax.ShapeDtypeStruct((VOCAB, CTX), jnp.float32),
    )(wt, b, gt, post)


def kernel(x, info_embedding, position_embedding, W, b):
    flat_t = _tc_detile(info_embedding.T)
    gt = _sc_embed_t(flat_t, x)
    out_t = _tc_unembed_t(W.T, b, gt, position_embedding.T)
    return out_t.T
